# baseline (speedup 1.0000x reference)
import functools

import jax
import jax.numpy as jnp
from jax import lax
from jax.experimental import pallas as pl
from jax.experimental.pallas import tpu as pltpu
from jax.experimental.pallas import tpu_sc as plsc

NC = 2
NS = 16
NW = NC * NS
LANE = 128


def _sc_segment_sum(table, src_flat, dst_flat, bpw0, bpw1, n_acc, n_out):
  mesh = plsc.VectorSubcoreMesh(
      core_axis_name="c", subcore_axis_name="s", num_cores=NC,
      num_subcores=NS)

  zrows = n_acc // NS
  orows = n_out // NS
  assert zrows % 8 == 0 and orows % 8 == 0 and n_out % 8 == 0
  assert bpw0 % 8 == 0 and bpw1 % 8 == 0

  zeros_acc = jnp.zeros((128, LANE), jnp.float32)
  zeros_cnt = jnp.zeros((zrows,), jnp.float32)
  ones_host = jnp.ones((LANE,), jnp.float32)

  @functools.partial(
      pl.kernel,
      out_type=[
          jax.ShapeDtypeStruct((NC, n_out, LANE), jnp.float32),
          jax.ShapeDtypeStruct((NC, n_out), jnp.float32),
      ],
      mesh=mesh,
      scratch_types=[
          pltpu.VMEM((bpw0, LANE), jnp.int32),
          pltpu.VMEM((bpw0, LANE), jnp.int32),
          pltpu.VMEM((2, LANE, LANE), jnp.float32),
          pltpu.VMEM((LANE,), jnp.float32),
          pltpu.VMEM((128, LANE), jnp.float32),
          pltpu.VMEM((zrows,), jnp.float32),
          pltpu.VMEM((n_out,), jnp.float32),
          pltpu.VMEM_SHARED((n_acc, LANE), jnp.float32),
          pltpu.VMEM_SHARED((n_acc,), jnp.float32),
          pltpu.SemaphoreType.DMA,
          pltpu.SemaphoreType.DMA,
      ],
  )
  def k(table_h, src_h, dst_h, zacc_h, zcnt_h, ones_h,
        part_h, cnt_h, src_v, dst_v, rows_v, ones_v, zb_v, zc_v, cb_v,
        acc_s, cnt_s, gsem, ssem):
    cid = lax.axis_index("c")
    sid = lax.axis_index("s")
    n_pairs = jnp.where(cid == 0, bpw0 // 2, bpw1 // 2)

    @pl.when(cid == 0)
    def _():
      pltpu.sync_copy(src_h.at[pl.ds(sid * bpw0, bpw0)], src_v)
      pltpu.sync_copy(dst_h.at[pl.ds(sid * bpw0, bpw0)], dst_v)

    @pl.when(cid == 1)
    def _():
      b = NS * bpw0 + sid * bpw1
      pltpu.sync_copy(src_h.at[pl.ds(b, bpw1)], src_v.at[pl.ds(0, bpw1)])
      pltpu.sync_copy(dst_h.at[pl.ds(b, bpw1)], dst_v.at[pl.ds(0, bpw1)])
    pltpu.sync_copy(ones_h, ones_v)
    pltpu.sync_copy(zacc_h, zb_v)
    pltpu.sync_copy(zcnt_h, zc_v)
    pltpu.sync_copy(zc_v, cnt_s.at[pl.ds(sid * zrows, zrows)])
    done = 0
    while done < zrows:
      step = min(128, zrows - done)
      pltpu.sync_copy(zb_v.at[pl.ds(0, step)],
                      acc_s.at[pl.ds(sid * zrows + done, step)])
      done += step

    plsc.subcore_barrier()

    def gather(j, buf):
      pltpu.async_copy(table_h.at[src_v.at[j]], rows_v.at[buf], gsem)

    def gather_wait(j, buf):
      pltpu.make_async_copy(table_h.at[src_v.at[j]], rows_v.at[buf],
                            gsem).wait()

    def scatter(j, buf):
      pltpu.async_copy(rows_v.at[buf], acc_s.at[dst_v.at[j]], ssem,
                       add=True)
      pltpu.async_copy(ones_v, cnt_s.at[dst_v.at[j]], ssem, add=True)

    def scatter_wait(j, buf):
      pltpu.make_async_copy(rows_v.at[buf], acc_s.at[dst_v.at[j]],
                            ssem).wait()
      pltpu.make_async_copy(ones_v, cnt_s.at[dst_v.at[j]], ssem).wait()

    gather(0, 0)

    def body(jj, carry):
      j0 = jj * 2
      j1 = j0 + 1
      gather_wait(j0, 0)
      gather(j1, 1)
      scatter(j0, 0)
      gather_wait(j1, 1)
      scatter_wait(j0, 0)

      @pl.when(jj + 1 < n_pairs)
      def _():
        gather(j0 + 2, 0)

      scatter(j1, 1)
      scatter_wait(j1, 1)
      return carry

    lax.fori_loop(0, n_pairs, body, 0, unroll=False)

    plsc.subcore_barrier()

    pltpu.sync_copy(acc_s.at[pl.ds(sid * orows, orows)],
                    rows_v.at[0, pl.ds(0, orows)])
    pltpu.sync_copy(rows_v.at[0, pl.ds(0, orows)],
                    part_h.at[cid, pl.ds(sid * orows, orows)])

    @pl.when(sid == 0)
    def _():
      pltpu.sync_copy(cnt_s.at[pl.ds(0, n_out)], cb_v)
      pltpu.sync_copy(cb_v, cnt_h.at[cid])

  return k(table, src_flat, dst_flat, zeros_acc, zeros_cnt, ones_host)


def _pad_edges(ei, batches_total, pad_lo, pad_hi):
  e_pad = NS * batches_total * LANE
  e = ei.shape[1]
  fill = pad_lo + jnp.arange(e_pad - e, dtype=jnp.int32) % (pad_hi - pad_lo)
  src = jnp.concatenate([ei[0], jnp.zeros((e_pad - e,), jnp.int32)])
  dst = jnp.concatenate([ei[1], fill])
  return src.reshape(-1, LANE), dst.reshape(-1, LANE)


def _tc_dense0(part, cnt, x_dst, wl, bl, wr):
  n, d = x_dst.shape

  def body(part_r, cnt_r, x_r, wl_r, bl_r, wr_r, out_r):
    agg = part_r[0] + part_r[1]
    denom = jnp.maximum(cnt_r[0] + cnt_r[1], 1.0)
    mean = agg / denom
    t = lax.dot_general(mean, wl_r[...], (((1,), (1,)), ((), ())),
                        preferred_element_type=jnp.float32)
    t2 = lax.dot_general(x_r[...], wr_r[...], (((1,), (1,)), ((), ())),
                         preferred_element_type=jnp.float32)
    out_r[...] = jnp.maximum(t + t2 + bl_r[...], 0.0)

  return pl.pallas_call(
      body,
      out_shape=jax.ShapeDtypeStruct((n, wl.shape[0]), jnp.float32),
  )(part, cnt.reshape(2, n, 1), x_dst, wl, bl.reshape(1, -1), wr)


def _tc_dense1(part, cnt, h_dst, wl, bl, wr):
  n = h_dst.shape[0]

  def body(part_r, cnt_r, h_r, wl_r, bl_r, wr_r, out_r):
    agg = part_r[0] + part_r[1]
    denom = jnp.maximum(cnt_r[0] + cnt_r[1], 1.0)
    mean = agg / denom
    t = lax.dot_general(mean, wl_r[...], (((1,), (1,)), ((), ())),
                        preferred_element_type=jnp.float32)
    t2 = lax.dot_general(h_r[...], wr_r[...], (((1,), (1,)), ((), ())),
                         preferred_element_type=jnp.float32)
    o = t + t2 + bl_r[...]
    m = jnp.max(o, axis=-1, keepdims=True)
    e = jnp.exp(o - m)
    lse = m + jnp.log(jnp.sum(e, axis=-1, keepdims=True))
    out_r[...] = o - lse

  return pl.pallas_call(
      body,
      out_shape=jax.ShapeDtypeStruct((n, wl.shape[0]), jnp.float32),
  )(part, cnt.reshape(2, n, 1), h_dst, wl, bl.reshape(1, -1), wr)


def kernel(x, edge_index_0, edge_index_1, size0_src, size0_dst,
           size1_src, size1_dst, W_l0, b_l0, W_r0, W_l1, b_l1, W_r1):
  n2 = 1000
  n_keep = 1024

  src0, dst0 = _pad_edges(edge_index_0, 160, n_keep, 5120)
  part0, cnt0 = _sc_segment_sum(x, src0, dst0, 80, 80, 5120, n_keep)
  h = _tc_dense0(part0, cnt0, x[:n_keep], W_l0, b_l0, W_r0)

  src1, dst1 = _pad_edges(edge_index_1, 80, n_keep, 2048)
  part1, cnt1 = _sc_segment_sum(h, src1, dst1, 40, 40, 2048, n_keep)
  o = _tc_dense1(part1, cnt1, h, W_l1, b_l1, W_r1)
  return o[:n2]


# speedup vs baseline: 3.2016x; 3.2016x over previous
import functools

import jax
import jax.numpy as jnp
from jax import lax
from jax.experimental import pallas as pl
from jax.experimental.pallas import tpu as pltpu
from jax.experimental.pallas import tpu_sc as plsc

NC = 2
NS = 16
NW = NC * NS
LANE = 128


def _sc_segment_sum(table, src_flat, dst_flat, bpw0, bpw1, n_acc, n_out):
  mesh = plsc.VectorSubcoreMesh(
      core_axis_name="c", subcore_axis_name="s", num_cores=NC,
      num_subcores=NS)

  zrows = n_acc // NS
  orows = n_out // NS
  assert zrows % 8 == 0 and orows % 8 == 0 and n_out % 8 == 0
  assert bpw0 % 8 == 0 and bpw1 % 8 == 0

  zeros_acc = jnp.zeros((128, LANE), jnp.float32)
  zeros_cnt = jnp.zeros((zrows,), jnp.float32)
  ones_host = jnp.ones((LANE,), jnp.float32)

  @functools.partial(
      pl.kernel,
      out_type=[
          jax.ShapeDtypeStruct((NC, n_out, LANE), jnp.float32),
          jax.ShapeDtypeStruct((NC, n_out), jnp.float32),
      ],
      mesh=mesh,
      scratch_types=[
          pltpu.VMEM((bpw0, LANE), jnp.int32),
          pltpu.VMEM((bpw0, LANE), jnp.int32),
          pltpu.VMEM((2, LANE, LANE), jnp.float32),
          pltpu.VMEM((LANE,), jnp.float32),
          pltpu.VMEM((128, LANE), jnp.float32),
          pltpu.VMEM((zrows,), jnp.float32),
          pltpu.VMEM((n_out,), jnp.float32),
          pltpu.VMEM_SHARED((n_acc, LANE), jnp.float32),
          pltpu.VMEM_SHARED((n_acc,), jnp.float32),
          pltpu.SemaphoreType.DMA,
          pltpu.SemaphoreType.DMA,
      ],
  )
  def k(table_h, src_h, dst_h, zacc_h, zcnt_h, ones_h,
        part_h, cnt_h, src_v, dst_v, rows_v, ones_v, zb_v, zc_v, cb_v,
        acc_s, cnt_s, gsem, ssem):
    cid = lax.axis_index("c")
    sid = lax.axis_index("s")
    n_pairs = jnp.where(cid == 0, bpw0 // 2, bpw1 // 2)

    @pl.when(cid == 0)
    def _():
      pltpu.sync_copy(src_h.at[pl.ds(sid * bpw0, bpw0)], src_v)
      pltpu.sync_copy(dst_h.at[pl.ds(sid * bpw0, bpw0)], dst_v)

    @pl.when(cid == 1)
    def _():
      b = NS * bpw0 + sid * bpw1
      pltpu.sync_copy(src_h.at[pl.ds(b, bpw1)], src_v.at[pl.ds(0, bpw1)])
      pltpu.sync_copy(dst_h.at[pl.ds(b, bpw1)], dst_v.at[pl.ds(0, bpw1)])
    pltpu.sync_copy(ones_h, ones_v)
    pltpu.sync_copy(zacc_h, zb_v)
    pltpu.sync_copy(zcnt_h, zc_v)
    pltpu.sync_copy(zc_v, cnt_s.at[pl.ds(sid * zrows, zrows)])
    done = 0
    while done < zrows:
      step = min(128, zrows - done)
      pltpu.sync_copy(zb_v.at[pl.ds(0, step)],
                      acc_s.at[pl.ds(sid * zrows + done, step)])
      done += step

    plsc.subcore_barrier()

    def gather(j, buf):
      pltpu.async_copy(table_h.at[src_v.at[j]], rows_v.at[buf], gsem)

    def gather_wait(j, buf):
      pltpu.make_async_copy(table_h.at[src_v.at[j]], rows_v.at[buf],
                            gsem).wait()

    def scatter(j, buf):
      pltpu.async_copy(rows_v.at[buf], acc_s.at[dst_v.at[j]], ssem,
                       add=True)
      pltpu.async_copy(ones_v, cnt_s.at[dst_v.at[j]], ssem, add=True)

    def scatter_wait(j, buf):
      pltpu.make_async_copy(rows_v.at[buf], acc_s.at[dst_v.at[j]],
                            ssem).wait()
      pltpu.make_async_copy(ones_v, cnt_s.at[dst_v.at[j]], ssem).wait()

    gather(0, 0)

    def body(jj, carry):
      j0 = jj * 2
      j1 = j0 + 1
      gather_wait(j0, 0)
      gather(j1, 1)
      scatter(j0, 0)
      gather_wait(j1, 1)
      scatter_wait(j0, 0)

      @pl.when(jj + 1 < n_pairs)
      def _():
        gather(j0 + 2, 0)

      scatter(j1, 1)
      scatter_wait(j1, 1)
      return carry

    lax.fori_loop(0, n_pairs, body, 0, unroll=False)

    plsc.subcore_barrier()

    pltpu.sync_copy(acc_s.at[pl.ds(sid * orows, orows)],
                    rows_v.at[0, pl.ds(0, orows)])
    pltpu.sync_copy(rows_v.at[0, pl.ds(0, orows)],
                    part_h.at[cid, pl.ds(sid * orows, orows)])

    @pl.when(sid == 0)
    def _():
      pltpu.sync_copy(cnt_s.at[pl.ds(0, n_out)], cb_v)
      pltpu.sync_copy(cb_v, cnt_h.at[cid])

  return k(table, src_flat, dst_flat, zeros_acc, zeros_cnt, ones_host)


def _pad_edges(ei, batches_total, pad_lo, pad_hi):
  e_pad = NS * batches_total * LANE
  e = ei.shape[1]
  ar = jnp.arange(e_pad - e, dtype=jnp.int32)
  src = jnp.concatenate([ei[0], ar % 1000])
  dst = jnp.concatenate([ei[1], pad_lo + ar % (pad_hi - pad_lo)])
  return src.reshape(-1, LANE), dst.reshape(-1, LANE)


def _tc_dense0(part, cnt, x_dst, wl, bl, wr):
  n, d = x_dst.shape

  def body(part_r, cnt_r, x_r, wl_r, bl_r, wr_r, out_r):
    agg = part_r[0] + part_r[1]
    denom = jnp.maximum(cnt_r[0] + cnt_r[1], 1.0)
    mean = agg / denom
    t = lax.dot_general(mean, wl_r[...], (((1,), (1,)), ((), ())),
                        preferred_element_type=jnp.float32)
    t2 = lax.dot_general(x_r[...], wr_r[...], (((1,), (1,)), ((), ())),
                         preferred_element_type=jnp.float32)
    out_r[...] = jnp.maximum(t + t2 + bl_r[...], 0.0)

  return pl.pallas_call(
      body,
      out_shape=jax.ShapeDtypeStruct((n, wl.shape[0]), jnp.float32),
  )(part, cnt.reshape(2, n, 1), x_dst, wl, bl.reshape(1, -1), wr)


def _tc_dense1(part, cnt, h_dst, wl, bl, wr):
  n = h_dst.shape[0]

  def body(part_r, cnt_r, h_r, wl_r, bl_r, wr_r, out_r):
    agg = part_r[0] + part_r[1]
    denom = jnp.maximum(cnt_r[0] + cnt_r[1], 1.0)
    mean = agg / denom
    t = lax.dot_general(mean, wl_r[...], (((1,), (1,)), ((), ())),
                        preferred_element_type=jnp.float32)
    t2 = lax.dot_general(h_r[...], wr_r[...], (((1,), (1,)), ((), ())),
                         preferred_element_type=jnp.float32)
    o = t + t2 + bl_r[...]
    m = jnp.max(o, axis=-1, keepdims=True)
    e = jnp.exp(o - m)
    lse = m + jnp.log(jnp.sum(e, axis=-1, keepdims=True))
    out_r[...] = o - lse

  return pl.pallas_call(
      body,
      out_shape=jax.ShapeDtypeStruct((n, wl.shape[0]), jnp.float32),
  )(part, cnt.reshape(2, n, 1), h_dst, wl, bl.reshape(1, -1), wr)


def kernel(x, edge_index_0, edge_index_1, size0_src, size0_dst,
           size1_src, size1_dst, W_l0, b_l0, W_r0, W_l1, b_l1, W_r1):
  n2 = 1000
  n_keep = 1024

  src0, dst0 = _pad_edges(edge_index_0, 160, n_keep, 5120)
  part0, cnt0 = _sc_segment_sum(x, src0, dst0, 80, 80, 5120, n_keep)
  h = _tc_dense0(part0, cnt0, x[:n_keep], W_l0, b_l0, W_r0)

  src1, dst1 = _pad_edges(edge_index_1, 80, n_keep, 2048)
  part1, cnt1 = _sc_segment_sum(h, src1, dst1, 40, 40, 2048, n_keep)
  o = _tc_dense1(part1, cnt1, h, W_l1, b_l1, W_r1)
  return o[:n2]


# speedup vs baseline: 3.9638x; 1.2381x over previous
import functools

import jax
import jax.numpy as jnp
from jax import lax
from jax.experimental import pallas as pl
from jax.experimental.pallas import tpu as pltpu
from jax.experimental.pallas import tpu_sc as plsc

NC = 2
NS = 16
NW = NC * NS
LANE = 128


def _sc_segment_sum(table, src_flat, dst_flat, bpw0, bpw1, n_acc, n_out):
  mesh = plsc.VectorSubcoreMesh(
      core_axis_name="c", subcore_axis_name="s", num_cores=NC,
      num_subcores=NS)

  zrows = n_acc // NS
  orows = n_out // NS
  assert zrows % 8 == 0 and orows % 8 == 0 and n_out % 8 == 0
  assert bpw0 % 8 == 0 and bpw1 % 8 == 0

  zeros_acc = jnp.zeros((128, LANE), jnp.float32)
  zeros_cnt = jnp.zeros((zrows,), jnp.float32)
  ones_host = jnp.ones((LANE,), jnp.float32)

  @functools.partial(
      pl.kernel,
      out_type=[
          jax.ShapeDtypeStruct((NC, n_out, LANE), jnp.float32),
          jax.ShapeDtypeStruct((NC, n_out), jnp.float32),
      ],
      mesh=mesh,
      scratch_types=[
          pltpu.VMEM((bpw0, LANE), jnp.int32),
          pltpu.VMEM((bpw0, LANE), jnp.int32),
          pltpu.VMEM((4, LANE, LANE), jnp.float32),
          pltpu.VMEM((LANE,), jnp.float32),
          pltpu.VMEM((zrows,), jnp.float32),
          pltpu.VMEM((n_out,), jnp.float32),
          pltpu.VMEM_SHARED((n_acc, LANE), jnp.float32),
          pltpu.VMEM_SHARED((n_acc,), jnp.float32),
          pltpu.SemaphoreType.DMA,
          pltpu.SemaphoreType.DMA,
      ],
  )
  def k(table_h, src_h, dst_h, zacc_h, zcnt_h, ones_h,
        part_h, cnt_h, src_v, dst_v, rows_v, ones_v, zc_v, cb_v,
        acc_s, cnt_s, gsem, ssem):
    cid = lax.axis_index("c")
    sid = lax.axis_index("s")
    @pl.when(cid == 0)
    def _():
      pltpu.sync_copy(src_h.at[pl.ds(sid * bpw0, bpw0)], src_v)
      pltpu.sync_copy(dst_h.at[pl.ds(sid * bpw0, bpw0)], dst_v)

    @pl.when(cid == 1)
    def _():
      b = NS * bpw0 + sid * bpw1
      pltpu.sync_copy(src_h.at[pl.ds(b, bpw1)], src_v.at[pl.ds(0, bpw1)])
      pltpu.sync_copy(dst_h.at[pl.ds(b, bpw1)], dst_v.at[pl.ds(0, bpw1)])
    pltpu.sync_copy(ones_h, ones_v)
    pltpu.sync_copy(zacc_h, rows_v.at[0])
    pltpu.sync_copy(zcnt_h, zc_v)
    pltpu.sync_copy(zc_v, cnt_s.at[pl.ds(sid * zrows, zrows)])
    done = 0
    while done < zrows:
      step = min(128, zrows - done)
      pltpu.sync_copy(rows_v.at[0, pl.ds(0, step)],
                      acc_s.at[pl.ds(sid * zrows + done, step)])
      done += step

    plsc.subcore_barrier()

    def gather(j, buf):
      pltpu.async_copy(table_h.at[src_v.at[j]], rows_v.at[buf], gsem)

    def gather_wait(j, buf):
      pltpu.make_async_copy(table_h.at[src_v.at[j]], rows_v.at[buf],
                            gsem).wait()

    def scatter(j, buf):
      pltpu.async_copy(rows_v.at[buf], acc_s.at[dst_v.at[j]], ssem,
                       add=True)
      pltpu.async_copy(ones_v, cnt_s.at[dst_v.at[j]], ssem, add=True)

    def scatter_wait(j, buf):
      pltpu.make_async_copy(rows_v.at[buf], acc_s.at[dst_v.at[j]],
                            ssem).wait()
      pltpu.make_async_copy(ones_v, cnt_s.at[dst_v.at[j]], ssem).wait()

    n_batches = jnp.where(cid == 0, bpw0, bpw1)
    n_quads = n_batches // 4

    gather(0, 0)
    gather(1, 1)
    gather(2, 2)

    def body(qq, carry):
      j = qq * 4
      gather(j + 3, 3)
      gather_wait(j, 0)
      scatter(j, 0)
      gather_wait(j + 1, 1)
      scatter(j + 1, 1)
      scatter_wait(j, 0)

      @pl.when(j + 4 < n_batches)
      def _():
        gather(j + 4, 0)

      gather_wait(j + 2, 2)
      scatter(j + 2, 2)
      scatter_wait(j + 1, 1)

      @pl.when(j + 5 < n_batches)
      def _():
        gather(j + 5, 1)

      gather_wait(j + 3, 3)
      scatter(j + 3, 3)
      scatter_wait(j + 2, 2)

      @pl.when(j + 6 < n_batches)
      def _():
        gather(j + 6, 2)

      scatter_wait(j + 3, 3)
      return carry

    lax.fori_loop(0, n_quads, body, 0, unroll=False)

    plsc.subcore_barrier()

    pltpu.sync_copy(acc_s.at[pl.ds(sid * orows, orows)],
                    rows_v.at[0, pl.ds(0, orows)])
    pltpu.sync_copy(rows_v.at[0, pl.ds(0, orows)],
                    part_h.at[cid, pl.ds(sid * orows, orows)])

    @pl.when(sid == 0)
    def _():
      pltpu.sync_copy(cnt_s.at[pl.ds(0, n_out)], cb_v)
      pltpu.sync_copy(cb_v, cnt_h.at[cid])

  return k(table, src_flat, dst_flat, zeros_acc, zeros_cnt, ones_host)


def _pad_edges(ei, batches_total, pad_lo, pad_hi):
  e_pad = NS * batches_total * LANE
  e = ei.shape[1]
  ar = jnp.arange(e_pad - e, dtype=jnp.int32)
  src = jnp.concatenate([ei[0], ar % 1000])
  dst = jnp.concatenate([ei[1], pad_lo + ar % (pad_hi - pad_lo)])
  return src.reshape(-1, LANE), dst.reshape(-1, LANE)


def _tc_dense0(part, cnt, x_dst, wl, bl, wr):
  n, d = x_dst.shape

  def body(part_r, cnt_r, x_r, wl_r, bl_r, wr_r, out_r):
    agg = part_r[0] + part_r[1]
    denom = jnp.maximum(cnt_r[0] + cnt_r[1], 1.0)
    mean = agg / denom
    t = lax.dot_general(mean, wl_r[...], (((1,), (1,)), ((), ())),
                        preferred_element_type=jnp.float32)
    t2 = lax.dot_general(x_r[...], wr_r[...], (((1,), (1,)), ((), ())),
                         preferred_element_type=jnp.float32)
    out_r[...] = jnp.maximum(t + t2 + bl_r[...], 0.0)

  return pl.pallas_call(
      body,
      out_shape=jax.ShapeDtypeStruct((n, wl.shape[0]), jnp.float32),
  )(part, cnt.reshape(2, n, 1), x_dst, wl, bl.reshape(1, -1), wr)


def _tc_dense1(part, cnt, h_dst, wl, bl, wr):
  n = h_dst.shape[0]

  def body(part_r, cnt_r, h_r, wl_r, bl_r, wr_r, out_r):
    agg = part_r[0] + part_r[1]
    denom = jnp.maximum(cnt_r[0] + cnt_r[1], 1.0)
    mean = agg / denom
    t = lax.dot_general(mean, wl_r[...], (((1,), (1,)), ((), ())),
                        preferred_element_type=jnp.float32)
    t2 = lax.dot_general(h_r[...], wr_r[...], (((1,), (1,)), ((), ())),
                         preferred_element_type=jnp.float32)
    o = t + t2 + bl_r[...]
    m = jnp.max(o, axis=-1, keepdims=True)
    e = jnp.exp(o - m)
    lse = m + jnp.log(jnp.sum(e, axis=-1, keepdims=True))
    out_r[...] = o - lse

  return pl.pallas_call(
      body,
      out_shape=jax.ShapeDtypeStruct((n, wl.shape[0]), jnp.float32),
  )(part, cnt.reshape(2, n, 1), h_dst, wl, bl.reshape(1, -1), wr)


def kernel(x, edge_index_0, edge_index_1, size0_src, size0_dst,
           size1_src, size1_dst, W_l0, b_l0, W_r0, W_l1, b_l1, W_r1):
  n2 = 1000
  n_keep = 1024

  src0, dst0 = _pad_edges(edge_index_0, 160, n_keep, 5120)
  part0, cnt0 = _sc_segment_sum(x, src0, dst0, 80, 80, 5120, n_keep)
  h = _tc_dense0(part0, cnt0, x[:n_keep], W_l0, b_l0, W_r0)

  src1, dst1 = _pad_edges(edge_index_1, 80, n_keep, 2048)
  part1, cnt1 = _sc_segment_sum(h, src1, dst1, 40, 40, 2048, n_keep)
  o = _tc_dense1(part1, cnt1, h, W_l1, b_l1, W_r1)
  return o[:n2]
